# Initial kernel scaffold; baseline (speedup 1.0000x reference)
import functools

import jax
import jax.numpy as jnp
from jax import lax
from jax.experimental import pallas as pl
from jax.experimental.pallas import tpu as pltpu
from jax.experimental.pallas import tpu_sc as plsc

B = 16384
F = 100
V = 1000000
NC = 2
NS = 16
NW = NC * NS
COLS_PER_W = B // NW
BLK = 128
NBLK = COLS_PER_W // BLK
G = BLK // 16


def _sc_body(idx_hbm, val_hbm, w_hbm, bias_hbm, out_hbm,
             idx_v, val_v, gat_v, out_v, bias_v, sem):
    wid = lax.axis_index("s") * NC + lax.axis_index("c")
    pltpu.sync_copy(bias_hbm, bias_v)

    def block(blk, carry):
        col0 = wid * COLS_PER_W + blk * BLK
        pltpu.sync_copy(idx_hbm.at[:, pl.ds(col0, BLK)], idx_v)
        pltpu.sync_copy(val_hbm.at[:, pl.ds(col0, BLK)], val_v)
        pltpu.async_copy(w_hbm.at[idx_v], gat_v, sem).wait()

        def fbody(f, accs):
            return tuple(
                accs[g] + gat_v[f, pl.ds(g * 16, 16)] * val_v[f, pl.ds(g * 16, 16)]
                for g in range(G)
            )

        accs = tuple(jnp.zeros((16,), jnp.float32) for _ in range(G))
        accs = lax.fori_loop(0, F, fbody, accs)
        b = bias_v[0]
        for g in range(G):
            out_v[pl.ds(g * 16, 16)] = accs[g] + b
        pltpu.sync_copy(out_v, out_hbm.at[pl.ds(col0, BLK)])
        return carry

    lax.fori_loop(0, NBLK, block, 0)


@jax.jit
def _sc_call(idx_t, val_t, w0, bias):
    mesh = plsc.VectorSubcoreMesh(core_axis_name="c", subcore_axis_name="s")
    f = pl.kernel(
        _sc_body,
        mesh=mesh,
        out_type=jax.ShapeDtypeStruct((B,), jnp.float32),
        scratch_types=[
            pltpu.VMEM((F, BLK), jnp.int32),
            pltpu.VMEM((F, BLK), jnp.float32),
            pltpu.VMEM((F, BLK), jnp.float32),
            pltpu.VMEM((BLK,), jnp.float32),
            pltpu.VMEM((1,), jnp.float32),
            pltpu.SemaphoreType.DMA,
        ],
    )
    return f(idx_t, val_t, w0, bias)


def kernel(index_list, value_list, W, bias):
    idx_t = index_list.T
    val_t = value_list.T
    w0 = W.reshape(V)
    res = _sc_call(idx_t, val_t, w0, bias)
    return res.reshape(B, 1)


# speedup vs baseline: 1.4184x; 1.4184x over previous
import functools

import jax
import jax.numpy as jnp
from jax import lax
from jax.experimental import pallas as pl
from jax.experimental.pallas import tpu as pltpu
from jax.experimental.pallas import tpu_sc as plsc

B = 16384
F = 100
V = 1000000
NC = 2
NS = 16
NW = NC * NS
COLS_PER_W = B // NW
BLK = 128
NBLK = COLS_PER_W // BLK
G = BLK // 16


def _sc_body(idx_hbm, val_hbm, w_hbm, bias_hbm, out_hbm,
             idx_v, val_v, gat_v, out_v, bias_v, sem):
    wid = lax.axis_index("s") * NC + lax.axis_index("c")
    pltpu.sync_copy(bias_hbm, bias_v)

    def block(blk, carry):
        col0 = wid * COLS_PER_W + blk * BLK
        pltpu.sync_copy(idx_hbm.at[:, pl.ds(col0, BLK)], idx_v)
        pltpu.sync_copy(val_hbm.at[:, pl.ds(col0, BLK)], val_v)

        def issue(f, c):
            pltpu.async_copy(w_hbm.at[idx_v.at[f]], gat_v.at[f], sem)
            return c

        lax.fori_loop(0, F, issue, 0)
        pltpu.make_async_copy(val_hbm.at[:, pl.ds(col0, BLK)], gat_v, sem).wait()

        def fbody(f, accs):
            return tuple(
                accs[g] + gat_v[f, pl.ds(g * 16, 16)] * val_v[f, pl.ds(g * 16, 16)]
                for g in range(G)
            )

        accs = tuple(jnp.zeros((16,), jnp.float32) for _ in range(G))
        accs = lax.fori_loop(0, F, fbody, accs)
        b = bias_v[...]
        for g in range(G):
            out_v[pl.ds(g * 16, 16)] = accs[g] + b
        pltpu.sync_copy(out_v, out_hbm.at[pl.ds(col0, BLK)])
        return carry

    lax.fori_loop(0, NBLK, block, 0)


@jax.jit
def _sc_call(idx_t, val_t, w0, bias):
    mesh = plsc.VectorSubcoreMesh(core_axis_name="c", subcore_axis_name="s")
    f = pl.kernel(
        _sc_body,
        mesh=mesh,
        out_type=jax.ShapeDtypeStruct((B,), jnp.float32),
        scratch_types=[
            pltpu.VMEM((F, BLK), jnp.int32),
            pltpu.VMEM((F, BLK), jnp.float32),
            pltpu.VMEM((F, BLK), jnp.float32),
            pltpu.VMEM((BLK,), jnp.float32),
            pltpu.VMEM((16,), jnp.float32),
            pltpu.SemaphoreType.DMA,
        ],
    )
    return f(idx_t, val_t, w0, bias)


def kernel(index_list, value_list, W, bias):
    idx_t = index_list.T
    val_t = value_list.T
    w0 = W.reshape(V)
    bias16 = jnp.broadcast_to(bias, (16,))
    res = _sc_call(idx_t, val_t, w0, bias16)
    return res.reshape(B, 1)
